# baseline (speedup 1.0000x reference)
import jax
import jax.numpy as jnp
from jax import lax
from jax.experimental import pallas as pl
from jax.experimental.pallas import tpu as pltpu
from jax.experimental.pallas import tpu_sc as plsc

N0 = 10000
N1 = 160000
D = 128
NEG_SLOPE = 0.2
LANES = 16
NSC = 2
NTILE = 16
BLK = 128
FEAT = 64
DACC = 4096
ZROWS = 1024


def _cdiv(a, b):
    return (a + b - 1) // b


def _mm_body(x_ref, w_ref, af_ref, ab_ref, y_ref, u_ref, v_ref):
    m = jnp.dot(x_ref[...], w_ref[...], preferred_element_type=jnp.float32)
    y_ref[...] = m
    u_ref[...] = jnp.dot(m, af_ref[...])
    v_ref[...] = jnp.dot(m, ab_ref[...])


def _mm(x, w, a):
    n0 = x.shape[0]
    br = 2048
    n = _cdiv(n0, br) * br
    x = jnp.concatenate([x, jnp.zeros((n - n0, D), jnp.float32)], axis=0)
    y, u, v = pl.pallas_call(
        _mm_body,
        grid=(n // br,),
        in_specs=[
            pl.BlockSpec((br, D), lambda i: (i, 0)),
            pl.BlockSpec((D, D), lambda i: (0, 0)),
            pl.BlockSpec((D,), lambda i: (0,)),
            pl.BlockSpec((D,), lambda i: (0,)),
        ],
        out_specs=[
            pl.BlockSpec((br, D), lambda i: (i, 0)),
            pl.BlockSpec((br,), lambda i: (i,)),
            pl.BlockSpec((br,), lambda i: (i,)),
        ],
        out_shape=[
            jax.ShapeDtypeStruct((n, D), jnp.float32),
            jax.ShapeDtypeStruct((n,), jnp.float32),
            jax.ShapeDtypeStruct((n,), jnp.float32),
        ],
    )(x, w, a[:D, 0], a[D:, 0])
    return y[:n0], u[:n0], v[:n0]


def _mean2_kernel(a_ref, b_ref, o_ref):
    o_ref[...] = (a_ref[...] + b_ref[...]) * 0.5


def _mean2(a, b):
    n = a.shape[0]
    blk = 2000
    return pl.pallas_call(
        _mean2_kernel,
        out_shape=jax.ShapeDtypeStruct(a.shape, a.dtype),
        grid=(n // blk,),
        in_specs=[
            pl.BlockSpec((blk, D), lambda i: (i, 0)),
            pl.BlockSpec((blk, D), lambda i: (i, 0)),
        ],
        out_specs=pl.BlockSpec((blk, D), lambda i: (i, 0)),
    )(a, b)


_PHASES = (
    (N0, N0, 320000),
    (N1, N1, 480000),
    (N0, N1, 320000),
    (N1, N0, 320000),
)


def _cfg(n_dst, n_src, n_edge):
    epad = _cdiv(n_edge, NTILE * BLK) * NTILE * BLK
    nacc = _cdiv(n_dst + 1, 16384) * 16384
    return dict(n_dst=n_dst, n_src=n_src, n_edge=n_edge, epad=epad, nacc=nacc,
                dchunks=nacc // DACC, etile=epad // NTILE,
                nblk=epad // NTILE // BLK, rows_per_tile=nacc // NTILE,
                drain_iters=nacc // NTILE // ZROWS)


_CFGS = tuple(_cfg(*p) for p in _PHASES)


def _build_sc_kernel():
    mesh = plsc.VectorSubcoreMesh(core_axis_name="c", subcore_axis_name="s")

    out_type = []
    for c in _CFGS:
        out_type += [
            jax.ShapeDtypeStruct((NSC, c["nacc"], FEAT), jnp.float32),
            jax.ShapeDtypeStruct((NSC * c["nacc"],), jnp.float32),
            jax.ShapeDtypeStruct((NSC, c["epad"]), jnp.float32),
            jax.ShapeDtypeStruct((NSC, c["epad"]), jnp.float32),
        ]
    max_nacc = max(c["nacc"] for c in _CFGS)

    scratch_types = [
        pltpu.VMEM((BLK,), jnp.int32),
        pltpu.VMEM((BLK,), jnp.int32),
        pltpu.VMEM((BLK,), jnp.int32),
        pltpu.VMEM((BLK,), jnp.int32),
        pltpu.VMEM((BLK,), jnp.float32),
        pltpu.VMEM((BLK,), jnp.float32),
        pltpu.VMEM((BLK,), jnp.float32),
        pltpu.VMEM((BLK,), jnp.float32),
        pltpu.VMEM((BLK, D), jnp.float32),
        pltpu.VMEM((BLK, FEAT), jnp.float32),
        pltpu.VMEM((DACC // NTILE, FEAT), jnp.float32),
        pltpu.VMEM((ZROWS,), jnp.float32),
        pltpu.VMEM_SHARED((max_nacc,), jnp.float32),
        pltpu.VMEM_SHARED((DACC, FEAT), jnp.float32),
        pltpu.SemaphoreType.DMA,
    ]

    def body(*refs):
        ins = refs[:22]
        outs = refs[22:38]
        (r_v, c_v, i2_v, i3_v, ug_v, vg_v, ea_v, rs_v, rows_v, sc_v,
         zb_v, z1_v, rowsum_s, accum_s, sem) = refs[38:]
        zr_hbm, z1_hbm = ins[20], ins[21]

        cid = lax.axis_index("c")
        tid = lax.axis_index("s")

        pltpu.sync_copy(z1_hbm, z1_v)

        for pi, cfg in enumerate(_CFGS):
            msg_hbm, u_hbm, v_hbm, r_hbm, c_hbm = ins[pi * 5:pi * 5 + 5]
            out_hbm, rsum_hbm, e_hbm, att_hbm = outs[pi * 4:pi * 4 + 4]
            nacc = cfg["nacc"]
            etile = cfg["etile"]
            nblk = cfg["nblk"]
            ebase = tid * etile
            abase = tid * cfg["rows_per_tile"]

            for k in range(cfg["drain_iters"]):
                pltpu.sync_copy(
                    z1_v, rowsum_s.at[pl.ds(abase + k * ZROWS, ZROWS)])
            plsc.subcore_barrier()

            def st_a(b, carry, ebase=ebase, r_hbm=r_hbm, c_hbm=c_hbm,
                     u_hbm=u_hbm, v_hbm=v_hbm, e_hbm=e_hbm, cid=cid):
                base = ebase + b * BLK
                pltpu.sync_copy(r_hbm.at[pl.ds(base, BLK)], r_v)
                pltpu.sync_copy(c_hbm.at[pl.ds(base, BLK)], c_v)
                pltpu.async_copy(u_hbm.at[r_v], ug_v, sem).wait()
                pltpu.async_copy(v_hbm.at[c_v], vg_v, sem).wait()
                for i in range(BLK // LANES):
                    s = pl.ds(i * LANES, LANES)
                    x = ug_v[s] + vg_v[s]
                    ea_v[s] = (jnp.maximum(x, 0.0)
                               + NEG_SLOPE * jnp.minimum(x, 0.0))
                pltpu.sync_copy(ea_v, e_hbm.at[cid, pl.ds(base, BLK)])
                pltpu.sync_copy(ea_v, rowsum_s.at[r_v], add=True)
                return carry

            lax.fori_loop(0, nblk, st_a, 0)
            plsc.subcore_barrier()

            for k in range(cfg["drain_iters"]):
                off = abase + k * ZROWS
                pltpu.sync_copy(rowsum_s.at[pl.ds(off, ZROWS)], z1_v)
                pltpu.sync_copy(
                    z1_v, rsum_hbm.at[pl.ds(cid * nacc + off, ZROWS)])
            plsc.subcore_barrier()
            pltpu.sync_copy(z1_hbm, z1_v)

            def st_a2(b, carry, ebase=ebase, r_hbm=r_hbm, e_hbm=e_hbm,
                      att_hbm=att_hbm, rsum_hbm=rsum_hbm, cid=cid, nacc=nacc):
                base = ebase + b * BLK
                pltpu.sync_copy(r_hbm.at[pl.ds(base, BLK)], r_v)
                pltpu.sync_copy(e_hbm.at[cid, pl.ds(base, BLK)], ea_v)
                for i in range(BLK // LANES):
                    s = pl.ds(i * LANES, LANES)
                    i2_v[s] = r_v[s] + cid * nacc
                pltpu.async_copy(rsum_hbm.at[i2_v], rs_v, sem).wait()
                for i in range(BLK // LANES):
                    s = pl.ds(i * LANES, LANES)
                    den = rs_v[s]
                    den = jnp.where(den == 0.0, 1.0, den)
                    ea_v[s] = ea_v[s] / den
                pltpu.sync_copy(ea_v, att_hbm.at[cid, pl.ds(base, BLK)])
                return carry

            lax.fori_loop(0, nblk, st_a2, 0)
            plsc.subcore_barrier()

            fbase = cid * FEAT

            def chunk_body(dk, carry0, ebase=ebase, r_hbm=r_hbm, c_hbm=c_hbm,
                           att_hbm=att_hbm, msg_hbm=msg_hbm, cid=cid,
                           out_hbm=out_hbm, fbase=fbase, nblk=nblk):
                doff = dk * DACC
                pltpu.sync_copy(zr_hbm, zb_v)
                pltpu.sync_copy(
                    zb_v,
                    accum_s.at[pl.ds(tid * (DACC // NTILE), DACC // NTILE), :])
                plsc.subcore_barrier()

                def st_b(b, carry):
                    base = ebase + b * BLK
                    pltpu.sync_copy(r_hbm.at[pl.ds(base, BLK)], r_v)
                    pltpu.sync_copy(c_hbm.at[pl.ds(base, BLK)], c_v)
                    pltpu.sync_copy(att_hbm.at[cid, pl.ds(base, BLK)], ea_v)
                    for i in range(BLK // LANES):
                        s = pl.ds(i * LANES, LANES)
                        loc = r_v[s] - doff
                        outside = ((loc >> 31) | ((DACC - 1 - loc) >> 31)) & 1
                        i3_v[s] = jnp.minimum(jnp.maximum(loc, 0), DACC - 1)
                        ea_v[s] = ea_v[s] * (1 - outside).astype(jnp.float32)
                    pltpu.async_copy(msg_hbm.at[c_v], rows_v, sem).wait()

                    def scale_g(g, carry2):
                        avec = ea_v[pl.ds(g * LANES, LANES)]
                        for jj in range(LANES):
                            j = g * LANES + jj
                            a = avec[jj]
                            for q in range(FEAT // LANES):
                                s = pl.ds(q * LANES, LANES)
                                sc_v[j, s] = a * rows_v[
                                    j, pl.ds(fbase + q * LANES, LANES)]
                        return carry2

                    lax.fori_loop(0, BLK // LANES, scale_g, 0)
                    pltpu.sync_copy(sc_v, accum_s.at[i3_v], add=True)
                    return carry

                lax.fori_loop(0, nblk, st_b, 0)
                plsc.subcore_barrier()
                pltpu.sync_copy(
                    accum_s.at[pl.ds(tid * (DACC // NTILE), DACC // NTILE), :],
                    zb_v)
                pltpu.sync_copy(
                    zb_v,
                    out_hbm.at[cid,
                               pl.ds(doff + tid * (DACC // NTILE),
                                     DACC // NTILE), :])
                plsc.subcore_barrier()
                return carry0

            lax.fori_loop(0, cfg["dchunks"], chunk_body, 0)

    return pl.kernel(body, mesh=mesh, out_type=tuple(out_type),
                     scratch_types=scratch_types)


_SC_KERNEL = _build_sc_kernel()


def _prep(msg, u, v, r, c, cfg):
    m = jnp.concatenate([msg, jnp.zeros((1, D), jnp.float32)], axis=0)
    up = jnp.concatenate([u, jnp.zeros((1,), jnp.float32)])
    vp = jnp.concatenate([v, jnp.zeros((1,), jnp.float32)])
    pad = cfg["epad"] - cfg["n_edge"]
    rp = jnp.concatenate([r, jnp.full((pad,), cfg["n_dst"], jnp.int32)])
    cp = jnp.concatenate([c, jnp.full((pad,), cfg["n_src"], jnp.int32)])
    return [m, up, vp, rp, cp]


def kernel(x_0, x_1, adjacency_0, adjacency_1, incidence_1_rows,
           incidence_1_cols, W0, a0, W1, a1, w_s, w_t, att_w):
    msg0, al0, be0 = _mm(x_0, W0, a0)
    msg1, al1, be1 = _mm(x_1, W1, a1)
    s_msg, s_a, s_b = _mm(x_1, w_s, att_w)
    t_msg, t_a, t_b = _mm(x_0, w_t, att_w)

    ins = (
        _prep(msg0, al0, be0, adjacency_0[0], adjacency_0[1], _CFGS[0])
        + _prep(msg1, al1, be1, adjacency_1[0], adjacency_1[1], _CFGS[1])
        + _prep(s_msg, t_a, s_b, incidence_1_rows, incidence_1_cols, _CFGS[2])
        + _prep(t_msg, s_a, t_b, incidence_1_cols, incidence_1_rows, _CFGS[3])
        + [jnp.zeros((DACC // NTILE, FEAT), jnp.float32),
           jnp.zeros((ZROWS,), jnp.float32)]
    )
    res = _SC_KERNEL(*ins)
    phases = []
    for pi, cfg in enumerate(_CFGS):
        out = res[pi * 4]
        phases.append(
            out.transpose(1, 0, 2)[:cfg["n_dst"]].reshape(cfg["n_dst"], D))

    return (_mean2(phases[0], phases[2]), _mean2(phases[1], phases[3]))
